# baseline (speedup 1.0000x reference)
import functools

import jax
import jax.numpy as jnp
from jax import lax
from jax.experimental import pallas as pl
from jax.experimental.pallas import tpu as pltpu
from jax.experimental.pallas import tpu_sc as plsc


def _make_sc_gather(V, D, B, S, NW, NB):
    mesh = plsc.VectorSubcoreMesh(core_axis_name="c", subcore_axis_name="s")
    info = plsc.get_sparse_core_info()
    NC = info.num_cores
    rows_per_w = B // NW
    n_outer = rows_per_w // NB

    @functools.partial(
        pl.kernel,
        mesh=mesh,
        compiler_params=pltpu.CompilerParams(use_tc_tiling_on_sc=False),
        out_type=jax.ShapeDtypeStruct((B, S, D), jnp.float32),
        scratch_types=[
            pltpu.VMEM((rows_per_w, S), jnp.int32),
            pltpu.VMEM((NB, S, D), jnp.float32),
            pltpu.SemaphoreType.DMA((NB,)),
            pltpu.SemaphoreType.DMA((NB,)),
        ],
    )
    def gather(idx_hbm, table_hbm, out_hbm, idx_v, rows_v, gsem, ssem):
        wid = lax.axis_index("s") * NC + lax.axis_index("c")
        base = wid * rows_per_w
        pltpu.sync_copy(idx_hbm.at[pl.ds(base, rows_per_w)], idx_v)

        def g_start(b, r):
            pltpu.async_copy(table_hbm.at[idx_v.at[r]], rows_v.at[b], gsem.at[b])

        def g_wait(b):
            pltpu.make_async_copy(
                table_hbm.at[idx_v.at[0]], rows_v.at[b], gsem.at[b]
            ).wait()

        def s_start(b, r):
            pltpu.async_copy(rows_v.at[b], out_hbm.at[base + r], ssem.at[b])

        def s_wait(b):
            pltpu.make_async_copy(
                rows_v.at[b], out_hbm.at[base], ssem.at[b]
            ).wait()

        for b in range(NB):
            g_start(b, b)

        def outer(o, carry):
            for b in range(NB):
                g_wait(b)
                s_start(b, o * NB + b)
            for b in range(NB):
                s_wait(b)

                @pl.when(o < n_outer - 1)
                def _():
                    g_start(b, o * NB + b + NB)

            return carry

        lax.fori_loop(0, n_outer, outer, 0)

    return gather


def kernel(input, table):
    B, S = input.shape
    V, D = table.shape
    NW = 32
    NB = 8

    out = _make_sc_gather(V, D, B, S, NW, NB)(input.astype(jnp.int32), table)
    return out


# speedup vs baseline: 1.0001x; 1.0001x over previous
import functools

import jax
import jax.numpy as jnp
from jax import lax
from jax.experimental import pallas as pl
from jax.experimental.pallas import tpu as pltpu
from jax.experimental.pallas import tpu_sc as plsc


def _make_sc_gather(V, D, NW, n_chunks, C):
    mesh = plsc.VectorSubcoreMesh(core_axis_name="c", subcore_axis_name="s")
    info = plsc.get_sparse_core_info()
    NC = info.num_cores
    per_w = n_chunks * C

    @functools.partial(
        pl.kernel,
        mesh=mesh,
        compiler_params=pltpu.CompilerParams(use_tc_tiling_on_sc=False),
        out_type=jax.ShapeDtypeStruct((NW * per_w, D), jnp.float32),
        scratch_types=[
            pltpu.VMEM((n_chunks, C), jnp.int32),
            pltpu.VMEM((2, C, D), jnp.float32),
            pltpu.SemaphoreType.DMA((2,)),
            pltpu.SemaphoreType.DMA((2,)),
        ],
    )
    def gather(idx_hbm, table_hbm, out_hbm, idx_v, rows_v, gsem, ssem):
        wid = lax.axis_index("s") * NC + lax.axis_index("c")
        base = wid * per_w
        pltpu.sync_copy(idx_hbm.at[wid], idx_v)

        def g_start(b, j):
            pltpu.async_copy(table_hbm.at[idx_v.at[j]], rows_v.at[b], gsem.at[b])

        def g_wait(b):
            pltpu.make_async_copy(
                table_hbm.at[idx_v.at[0]], rows_v.at[b], gsem.at[b]
            ).wait()

        def s_start(b, j):
            pltpu.async_copy(
                rows_v.at[b], out_hbm.at[pl.ds(base + j * C, C)], ssem.at[b]
            )

        def s_wait(b):
            pltpu.make_async_copy(
                rows_v.at[b], out_hbm.at[pl.ds(base, C)], ssem.at[b]
            ).wait()

        g_start(0, 0)
        g_start(1, 1)
        for j in range(n_chunks):
            b = j & 1
            g_wait(b)
            s_start(b, j)
            if j + 2 < n_chunks:
                s_wait(b)
                g_start(b, j + 2)
        s_wait(0)
        s_wait(1)

    return gather


def kernel(input, table):
    B, S = input.shape
    V, D = table.shape
    N = B * S
    NW = 32
    C = 800
    n_chunks = N // (NW * C)

    idx = input.reshape(NW, n_chunks, C).astype(jnp.int32)
    out = _make_sc_gather(V, D, NW, n_chunks, C)(idx, table)
    return out.reshape(B, S, D)


# speedup vs baseline: 1.0005x; 1.0004x over previous
import functools

import jax
import jax.numpy as jnp
from jax import lax
from jax.experimental import pallas as pl
from jax.experimental.pallas import tpu as pltpu
from jax.experimental.pallas import tpu_sc as plsc


def _make_sc_gather(V, D, NW, n_chunks, C, NB):
    mesh = plsc.VectorSubcoreMesh(core_axis_name="c", subcore_axis_name="s")
    info = plsc.get_sparse_core_info()
    NC = info.num_cores
    per_w = n_chunks * C
    n_outer = n_chunks // NB
    n_vecs = C // 16

    @functools.partial(
        pl.kernel,
        mesh=mesh,
        compiler_params=pltpu.CompilerParams(use_tc_tiling_on_sc=False),
        out_type=jax.ShapeDtypeStruct((NW * per_w, D), jnp.float32),
        scratch_types=[
            pltpu.VMEM((per_w,), jnp.int32),
            pltpu.VMEM((NB, C, D), jnp.float32),
            pltpu.SemaphoreType.DMA((NB,)),
            pltpu.SemaphoreType.DMA((NB,)),
        ],
    )
    def gather(idx_hbm, table_hbm, out_hbm, idx_v, rows_v, gsem, ssem):
        wid = lax.axis_index("s") * NC + lax.axis_index("c")
        base = wid * per_w
        pltpu.sync_copy(idx_hbm.at[wid], idx_v)

        def g_fire(b, j):
            def fire(k, carry):
                vec = idx_v[pl.ds(j * C + k * 16, 16)]
                pltpu.async_copy(
                    table_hbm.at[vec], rows_v.at[b, pl.ds(k * 16, 16)], gsem.at[b]
                )
                return carry

            lax.fori_loop(0, n_vecs, fire, 0)

        def g_wait(b):
            pltpu.make_async_copy(
                table_hbm.at[pl.ds(0, C)], rows_v.at[b], gsem.at[b]
            ).wait()

        def s_start(b, j):
            pltpu.async_copy(
                rows_v.at[b], out_hbm.at[pl.ds(base + j * C, C)], ssem.at[b]
            )

        def s_wait(b):
            pltpu.make_async_copy(
                rows_v.at[b], out_hbm.at[pl.ds(base, C)], ssem.at[b]
            ).wait()

        for b in range(NB):
            g_fire(b, b)

        def outer(o, carry):
            for b in range(NB):
                g_wait(b)
                s_start(b, o * NB + b)
            for b in range(NB):
                s_wait(b)

                @pl.when(o < n_outer - 1)
                def _():
                    g_fire(b, o * NB + b + NB)

            return carry

        lax.fori_loop(0, n_outer, outer, 0)

    return gather


def kernel(input, table):
    B, S = input.shape
    V, D = table.shape
    N = B * S
    NW = 32
    C = 256
    NB = 5
    n_chunks = N // (NW * C)

    idx = input.reshape(NW, N // NW).astype(jnp.int32)
    out = _make_sc_gather(V, D, NW, n_chunks, C, NB)(idx, table)
    return out.reshape(B, S, D)
